# baseline (speedup 1.0000x reference)
import functools

import jax
import jax.numpy as jnp
from jax import lax
from jax.experimental import pallas as pl
from jax.experimental.pallas import tpu as pltpu
from jax.experimental.pallas import tpu_sc as plsc

VOCAB = 1000000
DIM = 64
BATCH = 4096
HIST = 50
TOTAL = BATCH * HIST

NC = 2
NS = 16
NW = NC * NS
SAMPLES_PER_W = BATCH // NW
IDS_PER_W = SAMPLES_PER_W * HIST
SAMPLES_PER_CHUNK = 2
CHUNK = SAMPLES_PER_CHUNK * HIST
NCHUNKS = IDS_PER_W // CHUNK
LANES = 16
VPR = DIM // LANES


def _body(values_hbm, table_hbm, out_hbm, idx_v, rows0_v, rows1_v, acc_v,
          sem0, sem1):
    wid = lax.axis_index("s") * NC + lax.axis_index("c")

    pltpu.sync_copy(values_hbm.at[wid], idx_v)

    def _gather(c, rows, sem):
        return pltpu.async_copy(table_hbm.at[idx_v.at[c]], rows, sem)

    def _wait(rows, sem):
        pltpu.make_async_copy(table_hbm.at[idx_v.at[0]], rows, sem).wait()

    def _accum(c, rows):
        for s in range(SAMPLES_PER_CHUNK):
            r0 = s * HIST
            a = [rows[r0, pl.ds(j * LANES, LANES)] for j in range(VPR)]
            b = [rows[r0 + 1, pl.ds(j * LANES, LANES)] for j in range(VPR)]
            for r in range(2, HIST, 2):
                for j in range(VPR):
                    a[j] = a[j] + rows[r0 + r, pl.ds(j * LANES, LANES)]
                    b[j] = b[j] + rows[r0 + r + 1, pl.ds(j * LANES, LANES)]
            dst = (c * SAMPLES_PER_CHUNK + s) * DIM
            for j in range(VPR):
                acc_v[pl.ds(dst + j * LANES, LANES)] = a[j] + b[j]

    _gather(0, rows0_v, sem0)

    def _pair(cp, _):
        c0 = 2 * cp
        _wait(rows0_v, sem0)
        _gather(c0 + 1, rows1_v, sem1)
        _accum(c0, rows0_v)
        _wait(rows1_v, sem1)
        _gather(c0 + 2, rows0_v, sem0)
        _accum(c0 + 1, rows1_v)
        return 0

    lax.fori_loop(0, (NCHUNKS - 2) // 2, _pair, 0)

    _wait(rows0_v, sem0)
    _gather(NCHUNKS - 1, rows1_v, sem1)
    _accum(NCHUNKS - 2, rows0_v)
    _wait(rows1_v, sem1)
    _accum(NCHUNKS - 1, rows1_v)

    pltpu.sync_copy(acc_v, out_hbm.at[pl.ds(wid * SAMPLES_PER_W * DIM,
                                            SAMPLES_PER_W * DIM)])


@jax.jit
def _pooled_lookup(values, table):
    mesh = plsc.VectorSubcoreMesh(core_axis_name="c", subcore_axis_name="s")
    run = functools.partial(
        pl.kernel,
        out_type=jax.ShapeDtypeStruct((BATCH * DIM,), jnp.float32),
        mesh=mesh,
        compiler_params=pltpu.CompilerParams(use_tc_tiling_on_sc=False),
        scratch_types=[
            pltpu.VMEM((NCHUNKS, CHUNK), jnp.int32),
            pltpu.VMEM((CHUNK, DIM), jnp.float32),
            pltpu.VMEM((CHUNK, DIM), jnp.float32),
            pltpu.VMEM((SAMPLES_PER_W * DIM,), jnp.float32),
            pltpu.SemaphoreType.DMA,
            pltpu.SemaphoreType.DMA,
        ],
    )(_body)
    return run(values.reshape(NW, NCHUNKS, CHUNK), table).reshape(BATCH, DIM)


def kernel(values, lengths, table):
    del lengths
    return _pooled_lookup(values.astype(jnp.int32), table)


# speedup vs baseline: 1.3714x; 1.3714x over previous
import functools

import jax
import jax.numpy as jnp
from jax import lax
from jax.experimental import pallas as pl
from jax.experimental.pallas import tpu as pltpu
from jax.experimental.pallas import tpu_sc as plsc

VOCAB = 1000000
DIM = 64
BATCH = 4096
HIST = 50
TOTAL = BATCH * HIST

NC = 2
NS = 16
NW = NC * NS
SAMPLES_PER_W = BATCH // NW
IDS_PER_W = SAMPLES_PER_W * HIST
SAMPLES_PER_CHUNK = 2
CHUNK = SAMPLES_PER_CHUNK * HIST
NCHUNKS = IDS_PER_W // CHUNK
LANES = 16
CHUNK_PAD = 112
VPR = DIM // LANES


def _body(values_hbm, table_hbm, out_hbm, idx_v, rows0_v, rows1_v, acc_v,
          semg0a, semg0b, semg1a, semg1b):
    wid = lax.axis_index("s") * NC + lax.axis_index("c")

    pltpu.sync_copy(values_hbm.at[wid], idx_v)

    def _gather(c, rows, sema, semb):
        for g in range(CHUNK_PAD // LANES):
            vec = idx_v[c, pl.ds(g * LANES, LANES)]
            for l in range(min(LANES, CHUNK - g * LANES)):
                r = g * LANES + l
                pltpu.async_copy(table_hbm.at[pl.ds(vec[l], 1)],
                                 rows.at[pl.ds(r, 1)],
                                 sema if r % 2 == 0 else semb)

    def _gather_wait(rows, sema, semb):
        for r in range(CHUNK):
            pltpu.make_async_copy(table_hbm.at[pl.ds(0, 1)],
                                  rows.at[pl.ds(r, 1)],
                                  sema if r % 2 == 0 else semb).wait()

    def _accum(c, rows):
        for s in range(SAMPLES_PER_CHUNK):
            r0 = s * HIST
            a = [rows[r0, pl.ds(j * LANES, LANES)] for j in range(VPR)]
            b = [rows[r0 + 1, pl.ds(j * LANES, LANES)] for j in range(VPR)]
            for r in range(2, HIST, 2):
                for j in range(VPR):
                    a[j] = a[j] + rows[r0 + r, pl.ds(j * LANES, LANES)]
                    b[j] = b[j] + rows[r0 + r + 1, pl.ds(j * LANES, LANES)]
            smp = c * SAMPLES_PER_CHUNK + s
            for j in range(VPR):
                acc_v[smp, pl.ds(j * LANES, LANES)] = a[j] + b[j]

    _gather(0, rows0_v, semg0a, semg0b)
    _gather(1, rows1_v, semg1a, semg1b)

    def _pair(cp, _):
        c0 = 2 * cp
        _gather_wait(rows0_v, semg0a, semg0b)
        _accum(c0, rows0_v)
        _gather(c0 + 2, rows0_v, semg0a, semg0b)
        _gather_wait(rows1_v, semg1a, semg1b)
        _accum(c0 + 1, rows1_v)
        _gather(c0 + 3, rows1_v, semg1a, semg1b)
        return 0

    lax.fori_loop(0, (NCHUNKS - 2) // 2, _pair, 0)

    _gather_wait(rows0_v, semg0a, semg0b)
    _accum(NCHUNKS - 2, rows0_v)
    _gather_wait(rows1_v, semg1a, semg1b)
    _accum(NCHUNKS - 1, rows1_v)

    pltpu.sync_copy(acc_v, out_hbm.at[pl.ds(wid * SAMPLES_PER_W,
                                            SAMPLES_PER_W)])


@jax.jit
def _pooled_lookup(values, table):
    mesh = plsc.VectorSubcoreMesh(core_axis_name="c", subcore_axis_name="s")
    run = functools.partial(
        pl.kernel,
        out_type=jax.ShapeDtypeStruct((BATCH, DIM), jnp.float32),
        mesh=mesh,
        compiler_params=pltpu.CompilerParams(use_tc_tiling_on_sc=True),
        scratch_types=[
            pltpu.VMEM((NCHUNKS, CHUNK_PAD), jnp.int32),
            pltpu.VMEM((CHUNK, DIM), jnp.float32),
            pltpu.VMEM((CHUNK, DIM), jnp.float32),
            pltpu.VMEM((SAMPLES_PER_W, DIM), jnp.float32),
            pltpu.SemaphoreType.DMA,
            pltpu.SemaphoreType.DMA,
            pltpu.SemaphoreType.DMA,
            pltpu.SemaphoreType.DMA,
        ],
    )(_body)
    vals = jnp.pad(values.reshape(NW, NCHUNKS, CHUNK),
                   ((0, 0), (0, 0), (0, CHUNK_PAD - CHUNK)))
    return run(vals, table)


def kernel(values, lengths, table):
    del lengths
    return _pooled_lookup(values.astype(jnp.int32), table)
